# Initial kernel scaffold; baseline (speedup 1.0000x reference)
import jax
import jax.numpy as jnp
from jax.experimental import pallas as pl


def kernel(positions, pe):
    raise NotImplementedError("write your pallas kernel here")


# speedup vs baseline: 4.9601x; 4.9601x over previous
import functools

import jax
import jax.numpy as jnp
from jax import lax
from jax.experimental import pallas as pl
from jax.experimental.pallas import tpu as pltpu
from jax.experimental.pallas import tpu_sc as plsc

DIM = 64
ROWS = 16384
COLS = 200
B = ROWS * COLS
NC = 2
NS = 16
NW = NC * NS
BPW = B // NW
SUB = 128
NSUB = 8
CHUNK = SUB * NSUB
NITER = BPW // CHUNK
IDX_ROWS = B // SUB
IDX_RPW = BPW // SUB

_mesh = plsc.VectorSubcoreMesh(core_axis_name="c", subcore_axis_name="s")


@functools.partial(
    pl.kernel,
    mesh=_mesh,
    out_type=jax.ShapeDtypeStruct((B, DIM), jnp.float32),
    scratch_types=[
        pltpu.VMEM((NSUB, SUB), jnp.int32),
        pltpu.VMEM((CHUNK, DIM), jnp.float32),
        pltpu.SemaphoreType.DMA,
    ],
    compiler_params=pltpu.CompilerParams(use_tc_tiling_on_sc=False),
)
def _gather_kernel(idx_hbm, table_hbm, out_hbm, idx_v, rows_v, sem):
    wid = lax.axis_index("s") * NC + lax.axis_index("c")
    row_base = wid * BPW
    idx_base = wid * IDX_RPW

    def body(i, _):
        pltpu.sync_copy(idx_hbm.at[pl.ds(idx_base + i * NSUB, NSUB)], idx_v)
        handles = [
            pltpu.async_copy(
                table_hbm.at[idx_v.at[j]],
                rows_v.at[pl.ds(j * SUB, SUB)],
                sem,
            )
            for j in range(NSUB)
        ]
        for h in handles:
            h.wait()
        pltpu.sync_copy(rows_v, out_hbm.at[pl.ds(row_base + i * CHUNK, CHUNK)])
        return ()

    lax.fori_loop(0, NITER, body, (), unroll=False)


def kernel(positions, pe):
    idx2d = positions.reshape(IDX_ROWS, SUB)
    out = _gather_kernel(idx2d, pe)
    return out.reshape(positions.shape[0], positions.shape[1], pe.shape[1])


# speedup vs baseline: 5.1005x; 1.0283x over previous
import functools

import jax
import jax.numpy as jnp
from jax import lax
from jax.experimental import pallas as pl
from jax.experimental.pallas import tpu as pltpu
from jax.experimental.pallas import tpu_sc as plsc

DIM = 64
ROWS = 16384
COLS = 200
B = ROWS * COLS
NC = 2
NS = 16
NW = NC * NS
BPW = B // NW
SUB = 128
NSUB = 4
CHUNK = SUB * NSUB
NITER = BPW // CHUNK
NPAIR = NITER // 2
IDX_ROWS = B // SUB
IDX_RPW = BPW // SUB

_mesh = plsc.VectorSubcoreMesh(core_axis_name="c", subcore_axis_name="s")


@functools.partial(
    pl.kernel,
    mesh=_mesh,
    out_type=jax.ShapeDtypeStruct((B, DIM), jnp.float32),
    scratch_types=[
        pltpu.VMEM((2, NSUB, SUB), jnp.int32),
        pltpu.VMEM((2 * CHUNK, DIM), jnp.float32),
        pltpu.SemaphoreType.DMA,
        pltpu.SemaphoreType.DMA,
        pltpu.SemaphoreType.DMA,
        pltpu.SemaphoreType.DMA,
        pltpu.SemaphoreType.DMA,
    ],
    compiler_params=pltpu.CompilerParams(use_tc_tiling_on_sc=False),
)
def _gather_kernel(idx_hbm, table_hbm, out_hbm, idx_v, rows_v,
                   sem_i0, sem_i1, sem_g, sem_s0, sem_s1):
    wid = lax.axis_index("s") * NC + lax.axis_index("c")
    row_base = wid * BPW
    idx_base = wid * IDX_RPW
    sem_i = (sem_i0, sem_i1)
    sem_s = (sem_s0, sem_s1)

    def fire_idx(g, b):
        gg = jnp.minimum(g, NITER - 1)
        pltpu.async_copy(
            idx_hbm.at[pl.ds(idx_base + gg * NSUB, NSUB)], idx_v.at[b], sem_i[b])

    def wait_idx(b):
        pltpu.make_async_copy(
            idx_hbm.at[pl.ds(idx_base, NSUB)], idx_v.at[b], sem_i[b]).wait()

    def fire_gathers(g, b):
        for j in range(NSUB):
            pltpu.async_copy(
                table_hbm.at[idx_v.at[b, j]],
                rows_v.at[pl.ds((b * NSUB + j) * SUB, SUB)], sem_g)

    def drain_gathers(b):
        pltpu.make_async_copy(
            out_hbm.at[pl.ds(0, CHUNK)],
            rows_v.at[pl.ds(b * CHUNK, CHUNK)], sem_g).wait()

    def fire_store(g, b):
        pltpu.async_copy(
            rows_v.at[pl.ds(b * CHUNK, CHUNK)],
            out_hbm.at[pl.ds(row_base + g * CHUNK, CHUNK)], sem_s[b])

    def wait_store(b):
        pltpu.make_async_copy(
            out_hbm.at[pl.ds(0, CHUNK)],
            rows_v.at[pl.ds(b * CHUNK, CHUNK)], sem_s[b]).wait()

    fire_idx(0, 0)
    fire_idx(1, 1)
    wait_idx(0)
    fire_gathers(0, 0)
    wait_idx(1)
    drain_gathers(0)
    fire_idx(2, 0)
    fire_gathers(1, 1)
    fire_store(0, 0)

    def body(k, _):
        g0 = 2 * k
        wait_idx(0)
        drain_gathers(1)
        fire_idx(g0 + 1, 1)
        wait_store(0)
        fire_gathers(g0, 0)
        fire_store(g0 - 1, 1)
        wait_idx(1)
        drain_gathers(0)
        fire_idx(g0 + 2, 0)
        wait_store(1)
        fire_gathers(g0 + 1, 1)
        fire_store(g0, 0)
        return ()

    lax.fori_loop(1, NPAIR, body, (), unroll=False)

    wait_idx(0)
    drain_gathers(1)
    fire_store(NITER - 1, 1)
    wait_store(0)
    wait_store(1)


def kernel(positions, pe):
    idx2d = positions.reshape(IDX_ROWS, SUB)
    out = _gather_kernel(idx2d, pe)
    return out.reshape(positions.shape[0], positions.shape[1], pe.shape[1])


# speedup vs baseline: 5.1033x; 1.0005x over previous
import functools

import jax
import jax.numpy as jnp
from jax import lax
from jax.experimental import pallas as pl
from jax.experimental.pallas import tpu as pltpu
from jax.experimental.pallas import tpu_sc as plsc

DIM = 64
ROWS = 16384
COLS = 200
B = ROWS * COLS
NC = 2
NS = 16
NW = NC * NS
BPW = B // NW
CHUNK = 512
NITER = BPW // CHUNK
NPAIR = NITER // 2

_mesh = plsc.VectorSubcoreMesh(core_axis_name="c", subcore_axis_name="s")


@functools.partial(
    pl.kernel,
    mesh=_mesh,
    out_type=jax.ShapeDtypeStruct((B, DIM), jnp.float32),
    scratch_types=[
        pltpu.VMEM((2, CHUNK), jnp.int32),
        pltpu.VMEM((2, CHUNK, DIM), jnp.float32),
        pltpu.SemaphoreType.DMA,
        pltpu.SemaphoreType.DMA,
        pltpu.SemaphoreType.DMA,
        pltpu.SemaphoreType.DMA,
        pltpu.SemaphoreType.DMA,
    ],
    compiler_params=pltpu.CompilerParams(use_tc_tiling_on_sc=False),
)
def _gather_kernel(idx_hbm, table_hbm, out_hbm, idx_v, rows_v,
                   sem_i0, sem_i1, sem_g, sem_s0, sem_s1):
    wid = lax.axis_index("s") * NC + lax.axis_index("c")
    row_base = wid * BPW
    sem_i = (sem_i0, sem_i1)
    sem_s = (sem_s0, sem_s1)

    def fire_idx(g, b):
        gg = jnp.minimum(g, NITER - 1)
        pltpu.async_copy(
            idx_hbm.at[pl.ds(row_base + gg * CHUNK, CHUNK)], idx_v.at[b], sem_i[b])

    def wait_idx(b):
        pltpu.make_async_copy(
            idx_hbm.at[pl.ds(row_base, CHUNK)], idx_v.at[b], sem_i[b]).wait()

    def fire_gather(g, b):
        pltpu.async_copy(table_hbm.at[idx_v.at[b]], rows_v.at[b], sem_g)

    def drain_gather(b):
        pltpu.make_async_copy(
            out_hbm.at[pl.ds(0, CHUNK)], rows_v.at[b], sem_g).wait()

    def fire_store(g, b):
        pltpu.async_copy(
            rows_v.at[b],
            out_hbm.at[pl.ds(row_base + g * CHUNK, CHUNK)], sem_s[b])

    def wait_store(b):
        pltpu.make_async_copy(
            out_hbm.at[pl.ds(0, CHUNK)], rows_v.at[b], sem_s[b]).wait()

    fire_idx(0, 0)
    fire_idx(1, 1)
    wait_idx(0)
    fire_gather(0, 0)
    wait_idx(1)
    drain_gather(0)
    fire_idx(2, 0)
    fire_gather(1, 1)
    fire_store(0, 0)

    def body(k, _):
        g0 = 2 * k
        wait_idx(0)
        drain_gather(1)
        fire_idx(g0 + 1, 1)
        wait_store(0)
        fire_gather(g0, 0)
        fire_store(g0 - 1, 1)
        wait_idx(1)
        drain_gather(0)
        fire_idx(g0 + 2, 0)
        wait_store(1)
        fire_gather(g0 + 1, 1)
        fire_store(g0, 0)
        return ()

    lax.fori_loop(1, NPAIR, body, (), unroll=False)

    wait_idx(0)
    drain_gather(1)
    fire_store(NITER - 1, 1)
    wait_store(0)
    wait_store(1)


def kernel(positions, pe):
    idx1d = positions.reshape(B)
    out = _gather_kernel(idx1d, pe)
    return out.reshape(positions.shape[0], positions.shape[1], pe.shape[1])


# speedup vs baseline: 5.7985x; 1.1362x over previous
import functools

import jax
import jax.numpy as jnp
from jax import lax
from jax.experimental import pallas as pl
from jax.experimental.pallas import tpu as pltpu
from jax.experimental.pallas import tpu_sc as plsc

DIM = 64
ROWS = 16384
COLS = 200
B = ROWS * COLS
NC = 2
NS = 16
NW = NC * NS
BPW = B // NW
CHUNK = 512
NITER = BPW // CHUNK
NPAIR = NITER // 2

_mesh = plsc.VectorSubcoreMesh(core_axis_name="c", subcore_axis_name="s")


@functools.partial(
    pl.kernel,
    mesh=_mesh,
    out_type=jax.ShapeDtypeStruct((B, DIM), jnp.float32),
    scratch_types=[
        pltpu.VMEM((2, CHUNK), jnp.int32),
        pltpu.VMEM((2, CHUNK, DIM), jnp.float32),
        pltpu.VMEM_SHARED((8192, DIM), jnp.float32),
        pltpu.SemaphoreType.DMA,
        pltpu.SemaphoreType.DMA,
        pltpu.SemaphoreType.DMA,
        pltpu.SemaphoreType.DMA,
        pltpu.SemaphoreType.DMA,
    ],
    compiler_params=pltpu.CompilerParams(use_tc_tiling_on_sc=False),
)
def _gather_kernel(idx_hbm, table_hbm, out_hbm, idx_v, rows_v, table_spm,
                   sem_i0, sem_i1, sem_g, sem_s0, sem_s1):
    sid = lax.axis_index("s")
    wid = sid * NC + lax.axis_index("c")
    row_base = wid * BPW
    sem_i = (sem_i0, sem_i1)
    sem_s = (sem_s0, sem_s1)

    trows = 8192 // NS
    pltpu.sync_copy(table_hbm.at[pl.ds(sid * trows, trows)],
                    table_spm.at[pl.ds(sid * trows, trows)])
    plsc.subcore_barrier()

    def fire_idx(g, b):
        gg = jnp.minimum(g, NITER - 1)
        pltpu.async_copy(
            idx_hbm.at[pl.ds(row_base + gg * CHUNK, CHUNK)], idx_v.at[b], sem_i[b])

    def wait_idx(b):
        pltpu.make_async_copy(
            idx_hbm.at[pl.ds(row_base, CHUNK)], idx_v.at[b], sem_i[b]).wait()

    def fire_gather(g, b):
        pltpu.async_copy(table_spm.at[idx_v.at[b]], rows_v.at[b], sem_g)

    def drain_gather(b):
        pltpu.make_async_copy(
            out_hbm.at[pl.ds(0, CHUNK)], rows_v.at[b], sem_g).wait()

    def fire_store(g, b):
        pltpu.async_copy(
            rows_v.at[b],
            out_hbm.at[pl.ds(row_base + g * CHUNK, CHUNK)], sem_s[b])

    def wait_store(b):
        pltpu.make_async_copy(
            out_hbm.at[pl.ds(0, CHUNK)], rows_v.at[b], sem_s[b]).wait()

    fire_idx(0, 0)
    fire_idx(1, 1)
    wait_idx(0)
    fire_gather(0, 0)
    wait_idx(1)
    drain_gather(0)
    fire_idx(2, 0)
    fire_gather(1, 1)
    fire_store(0, 0)

    def body(k, _):
        g0 = 2 * k
        wait_idx(0)
        drain_gather(1)
        fire_idx(g0 + 1, 1)
        wait_store(0)
        fire_gather(g0, 0)
        fire_store(g0 - 1, 1)
        wait_idx(1)
        drain_gather(0)
        fire_idx(g0 + 2, 0)
        wait_store(1)
        fire_gather(g0 + 1, 1)
        fire_store(g0, 0)
        return ()

    lax.fori_loop(1, NPAIR, body, (), unroll=False)

    wait_idx(0)
    drain_gather(1)
    fire_store(NITER - 1, 1)
    wait_store(0)
    wait_store(1)


def kernel(positions, pe):
    idx1d = positions.reshape(B)
    out = _gather_kernel(idx1d, pe)
    return out.reshape(positions.shape[0], positions.shape[1], pe.shape[1])


# speedup vs baseline: 6.8007x; 1.1728x over previous
import functools

import jax
import jax.numpy as jnp
from jax import lax
from jax.experimental import pallas as pl
from jax.experimental.pallas import tpu as pltpu
from jax.experimental.pallas import tpu_sc as plsc

DIM = 64
PDIM = 128
ROWS = 16384
COLS = 200
B = ROWS * COLS
NC = 2
NS = 16
NW = NC * NS
BPW = B // NW
CHUNK = 256
NITER = BPW // CHUNK
NPAIR = NITER // 2

_mesh = plsc.VectorSubcoreMesh(core_axis_name="c", subcore_axis_name="s")


@functools.partial(
    pl.kernel,
    mesh=_mesh,
    out_type=jax.ShapeDtypeStruct((B, PDIM), jnp.float32),
    scratch_types=[
        pltpu.VMEM((2, CHUNK // 128, 128), jnp.int32),
        pltpu.VMEM((2, CHUNK, PDIM), jnp.float32),
        pltpu.SemaphoreType.DMA,
        pltpu.SemaphoreType.DMA,
        pltpu.SemaphoreType.DMA,
        pltpu.SemaphoreType.DMA,
        pltpu.SemaphoreType.DMA,
    ],
    compiler_params=pltpu.CompilerParams(use_tc_tiling_on_sc=True),
)
def _gather_kernel(idx_hbm, table_hbm, out_hbm, idx_v, rows_v,
                   sem_i0, sem_i1, sem_g, sem_s0, sem_s1):
    wid = lax.axis_index("s") * NC + lax.axis_index("c")
    row_base = wid * BPW
    sem_i = (sem_i0, sem_i1)
    sem_s = (sem_s0, sem_s1)

    irow_base = wid * (BPW // 128)
    nir = CHUNK // 128

    def fire_idx(g, b):
        gg = jnp.minimum(g, NITER - 1)
        pltpu.async_copy(
            idx_hbm.at[pl.ds(irow_base + gg * nir, nir)], idx_v.at[b], sem_i[b])

    def wait_idx(b):
        pltpu.make_async_copy(
            idx_hbm.at[pl.ds(irow_base, nir)], idx_v.at[b], sem_i[b]).wait()

    def fire_gather(g, b):
        for j in range(CHUNK // 128):
            pltpu.async_copy(table_hbm.at[idx_v.at[b, j]],
                             rows_v.at[b, pl.ds(j * 128, 128)], sem_g)

    def drain_gather(b):
        pltpu.make_async_copy(
            out_hbm.at[pl.ds(0, CHUNK)], rows_v.at[b], sem_g).wait()

    def fire_store(g, b):
        pltpu.async_copy(
            rows_v.at[b],
            out_hbm.at[pl.ds(row_base + g * CHUNK, CHUNK)], sem_s[b])

    def wait_store(b):
        pltpu.make_async_copy(
            out_hbm.at[pl.ds(0, CHUNK)], rows_v.at[b], sem_s[b]).wait()

    fire_idx(0, 0)
    fire_idx(1, 1)
    wait_idx(0)
    fire_gather(0, 0)
    wait_idx(1)
    drain_gather(0)
    fire_idx(2, 0)
    fire_gather(1, 1)
    fire_store(0, 0)

    def body(k, _):
        g0 = 2 * k
        wait_idx(0)
        drain_gather(1)
        fire_idx(g0 + 1, 1)
        wait_store(0)
        fire_gather(g0, 0)
        fire_store(g0 - 1, 1)
        wait_idx(1)
        drain_gather(0)
        fire_idx(g0 + 2, 0)
        wait_store(1)
        fire_gather(g0 + 1, 1)
        fire_store(g0, 0)
        return ()

    lax.fori_loop(1, NPAIR, body, (), unroll=False)

    wait_idx(0)
    drain_gather(1)
    fire_store(NITER - 1, 1)
    wait_store(0)
    wait_store(1)


def kernel(positions, pe):
    table128 = jnp.pad(pe, ((0, 0), (0, PDIM - DIM)))
    idx2d = positions.reshape(B // 128, 128)
    out = _gather_kernel(idx2d, table128)
    return out[:, :DIM].reshape(positions.shape[0], positions.shape[1], pe.shape[1])


# speedup vs baseline: 9.4209x; 1.3853x over previous
import functools

import jax
import jax.numpy as jnp
from jax import lax
from jax.experimental import pallas as pl
from jax.experimental.pallas import tpu as pltpu
from jax.experimental.pallas import tpu_sc as plsc

DIM = 64
PDIM = 128
ROWS = 16384
COLS = 200
B = ROWS * COLS
NC = 2
NS = 16
NW = NC * NS
BPW = B // NW
CHUNK = 128
NITER = BPW // CHUNK
NPAIR = NITER // 2

_mesh = plsc.VectorSubcoreMesh(core_axis_name="c", subcore_axis_name="s")


@functools.partial(
    pl.kernel,
    mesh=_mesh,
    out_type=jax.ShapeDtypeStruct((B, PDIM), jnp.float32),
    scratch_types=[
        pltpu.VMEM((2, CHUNK // 128, 128), jnp.int32),
        pltpu.VMEM((2, CHUNK, PDIM), jnp.float32),
        pltpu.VMEM_SHARED((8192, PDIM), jnp.float32),
        pltpu.SemaphoreType.DMA,
        pltpu.SemaphoreType.DMA,
        pltpu.SemaphoreType.DMA,
        pltpu.SemaphoreType.DMA,
        pltpu.SemaphoreType.DMA,
    ],
    compiler_params=pltpu.CompilerParams(use_tc_tiling_on_sc=True),
)
def _gather_kernel(idx_hbm, table_hbm, out_hbm, idx_v, rows_v, table_spm,
                   sem_i0, sem_i1, sem_g, sem_s0, sem_s1):
    sid = lax.axis_index("s")
    wid = sid * NC + lax.axis_index("c")
    row_base = wid * BPW
    sem_i = (sem_i0, sem_i1)
    sem_s = (sem_s0, sem_s1)

    trows = 8192 // NS
    pltpu.sync_copy(table_hbm.at[pl.ds(sid * trows, trows)],
                    table_spm.at[pl.ds(sid * trows, trows)])
    plsc.subcore_barrier()

    irow_base = wid * (BPW // 128)
    nir = CHUNK // 128

    def fire_idx(g, b):
        gg = jnp.minimum(g, NITER - 1)
        pltpu.async_copy(
            idx_hbm.at[pl.ds(irow_base + gg * nir, nir)], idx_v.at[b], sem_i[b])

    def wait_idx(b):
        pltpu.make_async_copy(
            idx_hbm.at[pl.ds(irow_base, nir)], idx_v.at[b], sem_i[b]).wait()

    def fire_gather(g, b):
        for j in range(CHUNK // 128):
            pltpu.async_copy(table_spm.at[idx_v.at[b, j]],
                             rows_v.at[b, pl.ds(j * 128, 128)], sem_g)

    def drain_gather(b):
        pltpu.make_async_copy(
            out_hbm.at[pl.ds(0, CHUNK)], rows_v.at[b], sem_g).wait()

    def fire_store(g, b):
        pltpu.async_copy(
            rows_v.at[b],
            out_hbm.at[pl.ds(row_base + g * CHUNK, CHUNK)], sem_s[b])

    def wait_store(b):
        pltpu.make_async_copy(
            out_hbm.at[pl.ds(0, CHUNK)], rows_v.at[b], sem_s[b]).wait()

    fire_idx(0, 0)
    fire_idx(1, 1)
    wait_idx(0)
    fire_gather(0, 0)
    wait_idx(1)
    drain_gather(0)
    fire_idx(2, 0)
    fire_gather(1, 1)
    fire_store(0, 0)

    def body(k, _):
        g0 = 2 * k
        wait_idx(0)
        drain_gather(1)
        fire_idx(g0 + 1, 1)
        wait_store(0)
        fire_gather(g0, 0)
        fire_store(g0 - 1, 1)
        wait_idx(1)
        drain_gather(0)
        fire_idx(g0 + 2, 0)
        wait_store(1)
        fire_gather(g0 + 1, 1)
        fire_store(g0, 0)
        return ()

    lax.fori_loop(1, NPAIR, body, (), unroll=False)

    wait_idx(0)
    drain_gather(1)
    fire_store(NITER - 1, 1)
    wait_store(0)
    wait_store(1)


def kernel(positions, pe):
    table128 = jnp.pad(pe, ((0, 0), (0, PDIM - DIM)))
    idx2d = positions.reshape(B // 128, 128)
    out = _gather_kernel(idx2d, table128)
    return out[:, :DIM].reshape(positions.shape[0], positions.shape[1], pe.shape[1])


# speedup vs baseline: 9.5649x; 1.0153x over previous
import functools

import jax
import jax.numpy as jnp
from jax import lax
from jax.experimental import pallas as pl
from jax.experimental.pallas import tpu as pltpu
from jax.experimental.pallas import tpu_sc as plsc

DIM = 64
PDIM = 128
ROWS = 16384
COLS = 200
B = ROWS * COLS
NC = 2
NS = 16
NW = NC * NS
BPW = B // NW
CHUNK = 128
NITER = BPW // CHUNK
NPAIR = NITER // 2

_mesh = plsc.VectorSubcoreMesh(core_axis_name="c", subcore_axis_name="s")


@functools.partial(
    pl.kernel,
    mesh=_mesh,
    out_type=jax.ShapeDtypeStruct((B, PDIM), jnp.float32),
    scratch_types=[
        pltpu.VMEM((2, CHUNK // 128, 128), jnp.int32),
        pltpu.VMEM((2, CHUNK, PDIM), jnp.float32),
        pltpu.VMEM_SHARED((8192, PDIM), jnp.float32),
        pltpu.SemaphoreType.DMA,
        pltpu.SemaphoreType.DMA,
        pltpu.SemaphoreType.DMA,
        pltpu.SemaphoreType.DMA,
        pltpu.SemaphoreType.DMA,
        pltpu.SemaphoreType.DMA,
    ],
    compiler_params=pltpu.CompilerParams(use_tc_tiling_on_sc=True),
)
def _gather_kernel(idx_hbm, table_hbm, out_hbm, idx_v, rows_v, table_spm,
                   sem_i0, sem_i1, sem_g0, sem_g1, sem_s0, sem_s1):
    sid = lax.axis_index("s")
    wid = sid * NC + lax.axis_index("c")
    row_base = wid * BPW
    sem_i = (sem_i0, sem_i1)
    sem_g = (sem_g0, sem_g1)
    sem_s = (sem_s0, sem_s1)

    trows = 8192 // NS
    pltpu.sync_copy(table_hbm.at[pl.ds(sid * trows, trows)],
                    table_spm.at[pl.ds(sid * trows, trows)])
    plsc.subcore_barrier()

    irow_base = wid * (BPW // 128)
    nir = CHUNK // 128

    def fire_idx(g, b):
        gg = jnp.minimum(g, NITER - 1)
        pltpu.async_copy(
            idx_hbm.at[pl.ds(irow_base + gg * nir, nir)], idx_v.at[b], sem_i[b])

    def wait_idx(b):
        pltpu.make_async_copy(
            idx_hbm.at[pl.ds(irow_base, nir)], idx_v.at[b], sem_i[b]).wait()

    def fire_gather(g, b):
        for j in range(CHUNK // 128):
            pltpu.async_copy(table_spm.at[idx_v.at[b, j]],
                             rows_v.at[b, pl.ds(j * 128, 128)], sem_g[b])

    def drain_gather(b):
        pltpu.make_async_copy(
            out_hbm.at[pl.ds(0, CHUNK)], rows_v.at[b], sem_g[b]).wait()

    def fire_store(g, b):
        pltpu.async_copy(
            rows_v.at[b],
            out_hbm.at[pl.ds(row_base + g * CHUNK, CHUNK)], sem_s[b])

    def wait_store(b):
        pltpu.make_async_copy(
            out_hbm.at[pl.ds(0, CHUNK)], rows_v.at[b], sem_s[b]).wait()

    fire_idx(0, 0)
    fire_idx(1, 1)
    wait_idx(0)
    fire_gather(0, 0)
    wait_idx(1)
    fire_gather(1, 1)
    drain_gather(0)
    fire_idx(2, 0)
    fire_store(0, 0)

    def body(k, _):
        g0 = 2 * k
        wait_idx(0)
        wait_store(0)
        fire_gather(g0, 0)
        drain_gather(1)
        fire_idx(g0 + 1, 1)
        fire_store(g0 - 1, 1)
        wait_idx(1)
        wait_store(1)
        fire_gather(g0 + 1, 1)
        drain_gather(0)
        fire_idx(g0 + 2, 0)
        fire_store(g0, 0)
        return ()

    lax.fori_loop(1, NPAIR, body, (), unroll=False)

    wait_idx(0)
    drain_gather(1)
    fire_store(NITER - 1, 1)
    wait_store(0)
    wait_store(1)


def kernel(positions, pe):
    table128 = jnp.pad(pe, ((0, 0), (0, PDIM - DIM)))
    idx2d = positions.reshape(B // 128, 128)
    out = _gather_kernel(idx2d, table128)
    return out[:, :DIM].reshape(positions.shape[0], positions.shape[1], pe.shape[1])
